# baseline (speedup 1.0000x reference)
import functools

import jax
import jax.numpy as jnp
from jax.experimental import pallas as pl


def _reduce_kernel(x_ref, o_ref, *, scale):
    m = pl.program_id(1)
    s = jnp.sum(x_ref[0, 0], axis=(1, 2)) * scale

    @pl.when(m == 0)
    def _init():
        o_ref[0, 0, :] = s

    @pl.when(m != 0)
    def _acc():
        o_ref[0, 0, :] += s


def kernel(x, protos):
    N, M, C, T, V = x.shape
    P = protos.shape[0]
    scale = 1.0 / (M * P)
    out = pl.pallas_call(
        functools.partial(_reduce_kernel, scale=scale),
        out_shape=jax.ShapeDtypeStruct((N, 1, C), x.dtype),
        grid=(N, M),
        in_specs=[pl.BlockSpec((1, 1, C, T, V), lambda n, m: (n, m, 0, 0, 0))],
        out_specs=pl.BlockSpec((1, 1, C), lambda n, m: (n, 0, 0)),
    )(x)
    return out.reshape(N, C)


# speedup vs baseline: 9.3248x; 9.3248x over previous
import functools

import jax
import jax.numpy as jnp
from jax.experimental import pallas as pl


def _reduce_kernel(x_ref, o_ref, *, scale):
    m = pl.program_id(1)
    s = jnp.sum(x_ref[0, 0], axis=(0, 1)) * scale

    @pl.when(m == 0)
    def _init():
        o_ref[0, 0, :] = s

    @pl.when(m != 0)
    def _acc():
        o_ref[0, 0, :] += s


def kernel(x, protos):
    N, M, C, T, V = x.shape
    P = protos.shape[0]
    scale = 1.0 / (M * P)
    xt = jnp.transpose(x, (0, 1, 4, 3, 2))
    out = pl.pallas_call(
        functools.partial(_reduce_kernel, scale=scale),
        out_shape=jax.ShapeDtypeStruct((N, 1, C), x.dtype),
        grid=(N, M),
        in_specs=[pl.BlockSpec((1, 1, V, T, C), lambda n, m: (n, m, 0, 0, 0))],
        out_specs=pl.BlockSpec((1, 1, C), lambda n, m: (n, 0, 0)),
    )(xt)
    return out.reshape(N, C)
